# baseline (speedup 1.0000x reference)
import functools

import jax
import jax.numpy as jnp
from jax import lax
from jax.experimental import pallas as pl
from jax.experimental.pallas import tpu as pltpu
from jax.experimental.pallas import tpu_sc as plsc

_Q = 2048
_D = 64
_NW = 32
_PPT = 64
_BI = 16

_mesh = plsc.VectorSubcoreMesh(core_axis_name="c", subcore_axis_name="s")


@functools.partial(
    pl.kernel,
    mesh=_mesh,
    out_type=jax.ShapeDtypeStruct((_Q, 2 * _D), jnp.float32),
    scratch_types=[
        pltpu.VMEM((_PPT,), jnp.int32),
        pltpu.VMEM((_PPT, 2 * _D), jnp.float32),
        pltpu.SemaphoreType.DMA,
    ],
)
def _sc_strip(pt_hbm, ps_hbm, idx, stage, sem):
    w = lax.axis_index("c") * 16 + lax.axis_index("s")
    base = w * _PPT
    for v in range(_PPT // 16):
        p = base + v * 16 + lax.iota(jnp.int32, 16)
        idx[pl.ds(v * 16, 16)] = jnp.clip(2 * p - 1533, 0, 1027)
    pltpu.async_copy(pt_hbm.at[idx], stage, sem).wait()
    pltpu.sync_copy(stage, ps_hbm.at[pl.ds(base, _PPT)])


_W = _Q + 128


def _tc_body(st_ref, o_ref):
    i0 = pl.program_id(0) * _BI
    for r in range(_BI):
        start = (_Q - 1) - (i0 + r)
        hi = lax.shift_right_logical(start, 7)
        lo = lax.bitwise_and(start, 127)
        win = st_ref[:, pl.ds(pl.multiple_of(hi * 128, 128), _W)]
        rolled = pltpu.roll(win, _W - lo, axis=1)
        o_ref[r] = rolled[:, :_Q]


_tc_expand = pl.pallas_call(
    _tc_body,
    grid=(_Q // _BI,),
    in_specs=[pl.BlockSpec((_D, 2 * _Q), lambda i: (0, 0))],
    out_specs=pl.BlockSpec((_BI, _D, _Q), lambda i: (i, 0, 0)),
    out_shape=jax.ShapeDtypeStruct((_Q, _D, _Q), jnp.float32),
)


def kernel(embedding, length_q, length_k):
    del length_q, length_k
    left = jnp.concatenate([embedding[:1], embedding], axis=0)
    right = jnp.concatenate([embedding, embedding[-1:]], axis=0)
    pair_table = jnp.concatenate([left, right], axis=1)
    pair_strip = _sc_strip(pair_table)
    strip_t = pair_strip.reshape(2 * _Q, _D).T
    out = _tc_expand(strip_t)
    return jnp.swapaxes(out, 1, 2)


# speedup vs baseline: 1.0367x; 1.0367x over previous
import functools

import jax
import jax.numpy as jnp
from jax import lax
from jax.experimental import pallas as pl
from jax.experimental.pallas import tpu as pltpu
from jax.experimental.pallas import tpu_sc as plsc

_Q = 2048
_D = 64
_NW = 32
_PPT = 64
_BI = 32

_mesh = plsc.VectorSubcoreMesh(core_axis_name="c", subcore_axis_name="s")


@functools.partial(
    pl.kernel,
    mesh=_mesh,
    out_type=jax.ShapeDtypeStruct((_Q, 2 * _D), jnp.float32),
    scratch_types=[
        pltpu.VMEM((_PPT,), jnp.int32),
        pltpu.VMEM((_PPT, 2 * _D), jnp.float32),
        pltpu.SemaphoreType.DMA,
    ],
)
def _sc_strip(pt_hbm, ps_hbm, idx, stage, sem):
    w = lax.axis_index("c") * 16 + lax.axis_index("s")
    base = w * _PPT
    for v in range(_PPT // 16):
        p = base + v * 16 + lax.iota(jnp.int32, 16)
        idx[pl.ds(v * 16, 16)] = jnp.clip(2 * p - 1533, 0, 1027)
    pltpu.async_copy(pt_hbm.at[idx], stage, sem).wait()
    pltpu.sync_copy(stage, ps_hbm.at[pl.ds(base, _PPT)])


_W = _Q + 128


def _tc_body(st_ref, o_ref):
    i0 = pl.program_id(0) * _BI
    for r in range(_BI):
        start = (_Q - 1) - (i0 + r)
        hi = lax.shift_right_logical(start, 7)
        lo = lax.bitwise_and(start, 127)
        win = st_ref[:, pl.ds(pl.multiple_of(hi * 128, 128), _W)]
        rolled = pltpu.roll(win, _W - lo, axis=1)
        o_ref[r] = rolled[:, :_Q]


_tc_expand = pl.pallas_call(
    _tc_body,
    grid=(_Q // _BI,),
    in_specs=[pl.BlockSpec((_D, 2 * _Q), lambda i: (0, 0))],
    out_specs=pl.BlockSpec((_BI, _D, _Q), lambda i: (i, 0, 0)),
    out_shape=jax.ShapeDtypeStruct((_Q, _D, _Q), jnp.float32),
)


def kernel(embedding, length_q, length_k):
    del length_q, length_k
    left = jnp.concatenate([embedding[:1], embedding], axis=0)
    right = jnp.concatenate([embedding, embedding[-1:]], axis=0)
    pair_table = jnp.concatenate([left, right], axis=1)
    pair_strip = _sc_strip(pair_table)
    strip_t = pair_strip.reshape(2 * _Q, _D).T
    out = _tc_expand(strip_t)
    return jnp.swapaxes(out, 1, 2)
